# baseline (speedup 1.0000x reference)
import functools

import jax
import jax.numpy as jnp
from jax import lax
from jax.experimental import pallas as pl
from jax.experimental.pallas import tpu as pltpu
from jax.experimental.pallas import tpu_sc as plsc

N_NODES = 10000
D = 128
N_EDGES = 320000

NC = 2
NS = 16
NW = NC * NS

RPT = 640
NPAD = NS * RPT
BLK = 64
NB = 160
NBC = 16
NBUF = 5
DBLK = 128
EPT = NB * BLK
EPAD = NW * EPT

_mesh = plsc.VectorSubcoreMesh(core_axis_name="c", subcore_axis_name="s")


@functools.partial(
    pl.kernel,
    mesh=_mesh,
    out_type=jax.ShapeDtypeStruct((NC, NPAD, D), jnp.float32),
    scratch_types=[
        pltpu.VMEM_SHARED((NPAD, D), jnp.float32),
        pltpu.VMEM((NBUF, BLK, D), jnp.float32),
        pltpu.VMEM((NBC, BLK), jnp.int32),
        pltpu.VMEM((NBC, BLK), jnp.int32),
        [pltpu.SemaphoreType.DMA] * NBUF,
    ],
)
def _scatter_kernel(xs_hbm, src_hbm, dst_hbm, out_hbm, acc, buf, src_v, dst_v, sems):
    c = lax.axis_index("c")
    s = lax.axis_index("s")
    base = s * RPT
    for k in range(RPT // BLK):
        sl = pl.ds(base + k * BLK, BLK)
        pltpu.sync_copy(xs_hbm.at[sl], buf.at[0])
        pltpu.sync_copy(buf.at[0], acc.at[sl])
    plsc.subcore_barrier()

    @pl.loop(0, NB, step=NBC)
    def _chunk(j0):
        pltpu.sync_copy(src_hbm.at[c, s, pl.ds(j0, NBC)], src_v)
        pltpu.sync_copy(dst_hbm.at[c, s, pl.ds(j0, NBC)], dst_v)
        for b in range(NBUF - 1):
            pltpu.async_copy(xs_hbm.at[src_v.at[b]], buf.at[b], sems[b])
        for i in range(NBC):
            bi = i % NBUF
            pltpu.make_async_copy(xs_hbm.at[src_v.at[i]], buf.at[bi], sems[bi]).wait()
            if i + NBUF - 1 < NBC:
                bn = (i + NBUF - 1) % NBUF
                pltpu.async_copy(xs_hbm.at[src_v.at[i + NBUF - 1]], buf.at[bn], sems[bn])
            pltpu.sync_copy(buf.at[bi], acc.at[dst_v.at[i]], add=True)

    plsc.subcore_barrier()
    for k in range(RPT // BLK):
        sl = pl.ds(base + k * BLK, BLK)
        pltpu.sync_copy(acc.at[sl], buf.at[0])
        pltpu.sync_copy(buf.at[0], out_hbm.at[c, sl])


DEPT = N_EDGES // NW
DSB = 80
DNB = DEPT // DSB


@functools.partial(
    pl.kernel,
    mesh=_mesh,
    out_type=jax.ShapeDtypeStruct((NC, NPAD, D), jnp.float32),
    scratch_types=[
        pltpu.VMEM_SHARED((NPAD, D), jnp.float32),
        pltpu.VMEM((DBLK, D), jnp.float32),
        pltpu.VMEM((DNB, DSB), jnp.int32),
    ],
)
def _deg_kernel(ones_hbm, dst_hbm, out_hbm, acc, buf, dst_v):
    c = lax.axis_index("c")
    s = lax.axis_index("s")
    base = s * RPT
    pltpu.sync_copy(ones_hbm, buf)
    for k in range(RPT // DBLK):
        pltpu.sync_copy(buf, acc.at[pl.ds(base + k * DBLK, DBLK)])
    pltpu.sync_copy(dst_hbm.at[c * NS + s], dst_v)
    plsc.subcore_barrier()

    @pl.loop(0, DNB)
    def _edges(j):
        pltpu.sync_copy(buf.at[pl.ds(0, DSB)], acc.at[dst_v.at[j]], add=True)

    plsc.subcore_barrier()
    for k in range(RPT // DBLK):
        sl = pl.ds(base + k * DBLK, DBLK)
        pltpu.sync_copy(acc.at[sl], buf)
        pltpu.sync_copy(buf, out_hbm.at[c, sl])


_ROWS = 1024
_GRID = NPAD // _ROWS


def _prep0_body(x_ref, w_ref, xw_ref):
    xw_ref[...] = jnp.dot(x_ref[...], w_ref[...], preferred_element_type=jnp.float32)


def _prep1_body(degp_ref, xw_ref, xs_ref, dinv_ref):
    deg = degp_ref[0, :, 0] + degp_ref[1, :, 0] - 1.0
    dinv = lax.rsqrt(deg)
    xs_ref[...] = xw_ref[...] * dinv[:, None]
    dinv_ref[...] = jnp.broadcast_to(dinv[:, None], dinv_ref.shape)


def _mid_body(dinv_ref, p_ref, xs_ref, b_ref, w_ref, out_ref):
    dinv = dinv_ref[:, 0]
    tot = p_ref[0] + p_ref[1] - xs_ref[...]
    h = jnp.maximum(tot * dinv[:, None] + b_ref[...], 0.0)
    out_ref[...] = jnp.dot(h, w_ref[...], preferred_element_type=jnp.float32) * dinv[:, None]


def _final_body(dinv_ref, p_ref, xs_ref, b_ref, out_ref):
    dinv = dinv_ref[:, 0]
    tot = p_ref[0] + p_ref[1] - xs_ref[...]
    out_ref[...] = tot * dinv[:, None] + b_ref[...]


_rows_spec = pl.BlockSpec((_ROWS, D), lambda i: (i, 0))
_dinv_spec = pl.BlockSpec((_ROWS, 16), lambda i: (i, 0))
_part_spec = pl.BlockSpec((NC, _ROWS, D), lambda i: (0, i, 0))
_degp_spec = pl.BlockSpec((NC, _ROWS, D), lambda i: (0, i, 0))
_w_spec = pl.BlockSpec((D, D), lambda i: (0, 0))
_b_spec = pl.BlockSpec((1, D), lambda i: (0, 0))
_f32 = lambda shape: jax.ShapeDtypeStruct(shape, jnp.float32)

_prep0 = pl.pallas_call(
    _prep0_body,
    grid=(_GRID,),
    in_specs=[_rows_spec, _w_spec],
    out_specs=_rows_spec,
    out_shape=_f32((NPAD, D)),
)

_prep1 = pl.pallas_call(
    _prep1_body,
    grid=(_GRID,),
    in_specs=[_degp_spec, _rows_spec],
    out_specs=[_rows_spec, _dinv_spec],
    out_shape=[_f32((NPAD, D)), _f32((NPAD, 16))],
)

_mid = pl.pallas_call(
    _mid_body,
    grid=(_GRID,),
    in_specs=[_dinv_spec, _part_spec, _rows_spec, _b_spec, _w_spec],
    out_specs=_rows_spec,
    out_shape=_f32((NPAD, D)),
)

_final = pl.pallas_call(
    _final_body,
    grid=(_GRID,),
    in_specs=[_dinv_spec, _part_spec, _rows_spec, _b_spec],
    out_specs=_rows_spec,
    out_shape=_f32((NPAD, D)),
)


@jax.jit
def kernel(x, edge_index, W1, b1, W2, b2):
    ei = edge_index.astype(jnp.int32)
    npadrows = NPAD - N_NODES
    pad_idx = N_NODES + (jnp.arange(EPAD - N_EDGES, dtype=jnp.int32) % npadrows)
    src = jnp.concatenate([ei[0], pad_idx]).reshape(NC, NS, NB, BLK)
    dst = jnp.concatenate([ei[1], pad_idx]).reshape(NC, NS, NB, BLK)
    dst_d = ei[1].reshape(NW, DNB, DSB)
    x_pad = jnp.pad(x, ((0, npadrows), (0, 0)))
    ones_tab = jnp.ones((DBLK, D), jnp.float32)

    degp = _deg_kernel(ones_tab, dst_d)
    xw1 = _prep0(x_pad, W1)
    xs1, dinv = _prep1(degp, xw1)
    p1 = _scatter_kernel(xs1, src, dst)
    xs2 = _mid(dinv, p1, xs1, b1.reshape(1, D), W2)
    p2 = _scatter_kernel(xs2, src, dst)
    out = _final(dinv, p2, xs2, b2.reshape(1, D))
    return out[:N_NODES]


# speedup vs baseline: 1.0135x; 1.0135x over previous
import functools

import jax
import jax.numpy as jnp
from jax import lax
from jax.experimental import pallas as pl
from jax.experimental.pallas import tpu as pltpu
from jax.experimental.pallas import tpu_sc as plsc

N_NODES = 10000
D = 128
N_EDGES = 320000

NC = 2
NS = 16
NW = NC * NS

RPT = 640
NPAD = NS * RPT
BLK = 64
NB = 160
NBC = 16
NBUF = 4
DBLK = 128
EPT = NB * BLK
EPAD = NW * EPT

_mesh = plsc.VectorSubcoreMesh(core_axis_name="c", subcore_axis_name="s")


@functools.partial(
    pl.kernel,
    mesh=_mesh,
    out_type=jax.ShapeDtypeStruct((NC, NPAD, D), jnp.float32),
    scratch_types=[
        pltpu.VMEM_SHARED((NPAD, D), jnp.float32),
        pltpu.VMEM((NBUF, BLK, D), jnp.float32),
        pltpu.VMEM((NBC, BLK), jnp.int32),
        pltpu.VMEM((NBC, BLK), jnp.int32),
        [pltpu.SemaphoreType.DMA] * NBUF,
    ],
)
def _scatter_kernel(xs_hbm, src_hbm, dst_hbm, out_hbm, acc, buf, src_v, dst_v, sems):
    c = lax.axis_index("c")
    s = lax.axis_index("s")
    base = s * RPT
    for k in range(RPT // BLK):
        sl = pl.ds(base + k * BLK, BLK)
        pltpu.sync_copy(xs_hbm.at[sl], buf.at[0])
        pltpu.sync_copy(buf.at[0], acc.at[sl])
    plsc.subcore_barrier()

    @pl.loop(0, NB, step=NBC)
    def _chunk(j0):
        pltpu.sync_copy(src_hbm.at[c, s, pl.ds(j0, NBC)], src_v)
        pltpu.sync_copy(dst_hbm.at[c, s, pl.ds(j0, NBC)], dst_v)
        for b in range(NBUF - 1):
            pltpu.async_copy(xs_hbm.at[src_v.at[b]], buf.at[b], sems[b])
        for i in range(NBC):
            bi = i % NBUF
            pltpu.make_async_copy(xs_hbm.at[src_v.at[i]], buf.at[bi], sems[bi]).wait()
            if i + NBUF - 1 < NBC:
                bn = (i + NBUF - 1) % NBUF
                pltpu.async_copy(xs_hbm.at[src_v.at[i + NBUF - 1]], buf.at[bn], sems[bn])
            pltpu.sync_copy(buf.at[bi], acc.at[dst_v.at[i]], add=True)

    plsc.subcore_barrier()
    for k in range(RPT // BLK):
        sl = pl.ds(base + k * BLK, BLK)
        pltpu.sync_copy(acc.at[sl], buf.at[0])
        pltpu.sync_copy(buf.at[0], out_hbm.at[c, sl])


DEPT = N_EDGES // NW
DSB = 80
DNB = DEPT // DSB


@functools.partial(
    pl.kernel,
    mesh=_mesh,
    out_type=jax.ShapeDtypeStruct((NC, NPAD, D), jnp.float32),
    scratch_types=[
        pltpu.VMEM_SHARED((NPAD, D), jnp.float32),
        pltpu.VMEM((DBLK, D), jnp.float32),
        pltpu.VMEM((DNB, DSB), jnp.int32),
    ],
)
def _deg_kernel(ones_hbm, dst_hbm, out_hbm, acc, buf, dst_v):
    c = lax.axis_index("c")
    s = lax.axis_index("s")
    base = s * RPT
    pltpu.sync_copy(ones_hbm, buf)
    for k in range(RPT // DBLK):
        pltpu.sync_copy(buf, acc.at[pl.ds(base + k * DBLK, DBLK)])
    pltpu.sync_copy(dst_hbm.at[c * NS + s], dst_v)
    plsc.subcore_barrier()

    @pl.loop(0, DNB)
    def _edges(j):
        pltpu.sync_copy(buf.at[pl.ds(0, DSB)], acc.at[dst_v.at[j]], add=True)

    plsc.subcore_barrier()
    for k in range(RPT // DBLK):
        sl = pl.ds(base + k * DBLK, DBLK)
        pltpu.sync_copy(acc.at[sl], buf)
        pltpu.sync_copy(buf, out_hbm.at[c, sl])


_ROWS = 1024
_GRID = NPAD // _ROWS


def _prep0_body(x_ref, w_ref, xw_ref):
    xw_ref[...] = jnp.dot(x_ref[...], w_ref[...], preferred_element_type=jnp.float32)


def _prep1_body(degp_ref, xw_ref, xs_ref, dinv_ref):
    deg = degp_ref[0, :, 0] + degp_ref[1, :, 0] - 1.0
    dinv = lax.rsqrt(deg)
    xs_ref[...] = xw_ref[...] * dinv[:, None]
    dinv_ref[...] = jnp.broadcast_to(dinv[:, None], dinv_ref.shape)


def _mid_body(dinv_ref, p_ref, xs_ref, b_ref, w_ref, out_ref):
    dinv = dinv_ref[:, 0]
    tot = p_ref[0] + p_ref[1] - xs_ref[...]
    h = jnp.maximum(tot * dinv[:, None] + b_ref[...], 0.0)
    out_ref[...] = jnp.dot(h, w_ref[...], preferred_element_type=jnp.float32) * dinv[:, None]


def _final_body(dinv_ref, p_ref, xs_ref, b_ref, out_ref):
    dinv = dinv_ref[:, 0]
    tot = p_ref[0] + p_ref[1] - xs_ref[...]
    out_ref[...] = tot * dinv[:, None] + b_ref[...]


_rows_spec = pl.BlockSpec((_ROWS, D), lambda i: (i, 0))
_dinv_spec = pl.BlockSpec((_ROWS, 16), lambda i: (i, 0))
_part_spec = pl.BlockSpec((NC, _ROWS, D), lambda i: (0, i, 0))
_degp_spec = pl.BlockSpec((NC, _ROWS, D), lambda i: (0, i, 0))
_w_spec = pl.BlockSpec((D, D), lambda i: (0, 0))
_b_spec = pl.BlockSpec((1, D), lambda i: (0, 0))
_f32 = lambda shape: jax.ShapeDtypeStruct(shape, jnp.float32)

_prep0 = pl.pallas_call(
    _prep0_body,
    grid=(_GRID,),
    in_specs=[_rows_spec, _w_spec],
    out_specs=_rows_spec,
    out_shape=_f32((NPAD, D)),
)

_prep1 = pl.pallas_call(
    _prep1_body,
    grid=(_GRID,),
    in_specs=[_degp_spec, _rows_spec],
    out_specs=[_rows_spec, _dinv_spec],
    out_shape=[_f32((NPAD, D)), _f32((NPAD, 16))],
)

_mid = pl.pallas_call(
    _mid_body,
    grid=(_GRID,),
    in_specs=[_dinv_spec, _part_spec, _rows_spec, _b_spec, _w_spec],
    out_specs=_rows_spec,
    out_shape=_f32((NPAD, D)),
)

_final = pl.pallas_call(
    _final_body,
    grid=(_GRID,),
    in_specs=[_dinv_spec, _part_spec, _rows_spec, _b_spec],
    out_specs=_rows_spec,
    out_shape=_f32((NPAD, D)),
)


@jax.jit
def kernel(x, edge_index, W1, b1, W2, b2):
    ei = edge_index.astype(jnp.int32)
    npadrows = NPAD - N_NODES
    pad_idx = N_NODES + (jnp.arange(EPAD - N_EDGES, dtype=jnp.int32) % npadrows)
    src = jnp.concatenate([ei[0], pad_idx]).reshape(NC, NS, NB, BLK)
    dst = jnp.concatenate([ei[1], pad_idx]).reshape(NC, NS, NB, BLK)
    dst_d = ei[1].reshape(NW, DNB, DSB)
    x_pad = jnp.pad(x, ((0, npadrows), (0, 0)))
    ones_tab = jnp.ones((DBLK, D), jnp.float32)

    degp = _deg_kernel(ones_tab, dst_d)
    xw1 = _prep0(x_pad, W1)
    xs1, dinv = _prep1(degp, xw1)
    p1 = _scatter_kernel(xs1, src, dst)
    xs2 = _mid(dinv, p1, xs1, b1.reshape(1, D), W2)
    p2 = _scatter_kernel(xs2, src, dst)
    out = _final(dinv, p2, xs2, b2.reshape(1, D))
    return out[:N_NODES]


# speedup vs baseline: 1.0703x; 1.0561x over previous
import functools

import jax
import jax.numpy as jnp
from jax import lax
from jax.experimental import pallas as pl
from jax.experimental.pallas import tpu as pltpu
from jax.experimental.pallas import tpu_sc as plsc

N_NODES = 10000
D = 128
N_EDGES = 320000

NC = 2
NS = 16
NW = NC * NS

RPT = 640
NPAD = NS * RPT
BLK = 64
NB = 160
NBC = 16
NBUF = 4
DBLK = 128
EPT = NB * BLK
EPAD = NW * EPT

_mesh = plsc.VectorSubcoreMesh(core_axis_name="c", subcore_axis_name="s")


@functools.partial(
    pl.kernel,
    mesh=_mesh,
    out_type=jax.ShapeDtypeStruct((NC, NPAD, D), jnp.float32),
    scratch_types=[
        pltpu.VMEM_SHARED((NPAD, D), jnp.float32),
        pltpu.VMEM((NBUF, BLK, D), jnp.float32),
        pltpu.VMEM((NBC, BLK), jnp.int32),
        pltpu.VMEM((NBC, BLK), jnp.int32),
        [pltpu.SemaphoreType.DMA] * NBUF,
    ],
)
def _scatter_kernel(xs_hbm, src_hbm, dst_hbm, out_hbm, acc, buf, src_v, dst_v, sems):
    c = lax.axis_index("c")
    s = lax.axis_index("s")
    base = s * RPT
    NIK = RPT // BLK
    for b in range(min(NBUF, NIK)):
        pltpu.async_copy(xs_hbm.at[pl.ds(base + b * BLK, BLK)], buf.at[b], sems[b])
    for k in range(NIK):
        b = k % NBUF
        sl = pl.ds(base + k * BLK, BLK)
        pltpu.make_async_copy(xs_hbm.at[sl], buf.at[b], sems[b]).wait()
        pltpu.sync_copy(buf.at[b], acc.at[sl])
        if k + NBUF < NIK:
            nsl = pl.ds(base + (k + NBUF) * BLK, BLK)
            pltpu.async_copy(xs_hbm.at[nsl], buf.at[b], sems[b])
    plsc.subcore_barrier()

    @pl.loop(0, NB, step=NBC)
    def _chunk(j0):
        pltpu.sync_copy(src_hbm.at[c, s, pl.ds(j0, NBC)], src_v)
        pltpu.sync_copy(dst_hbm.at[c, s, pl.ds(j0, NBC)], dst_v)
        for b in range(NBUF - 1):
            pltpu.async_copy(xs_hbm.at[src_v.at[b]], buf.at[b], sems[b])
        for i in range(NBC):
            bi = i % NBUF
            pltpu.make_async_copy(xs_hbm.at[src_v.at[i]], buf.at[bi], sems[bi]).wait()
            if i + NBUF - 1 < NBC:
                bn = (i + NBUF - 1) % NBUF
                pltpu.async_copy(xs_hbm.at[src_v.at[i + NBUF - 1]], buf.at[bn], sems[bn])
            pltpu.sync_copy(buf.at[bi], acc.at[dst_v.at[i]], add=True)

    plsc.subcore_barrier()
    for k in range(NIK):
        b = k % NBUF
        sl = pl.ds(base + k * BLK, BLK)
        if k >= NBUF:
            psl = pl.ds(base + (k - NBUF) * BLK, BLK)
            pltpu.make_async_copy(buf.at[b], out_hbm.at[c, psl], sems[b]).wait()
        pltpu.sync_copy(acc.at[sl], buf.at[b])
        pltpu.async_copy(buf.at[b], out_hbm.at[c, sl], sems[b])
    for k in range(NIK - NBUF, NIK):
        b = k % NBUF
        sl = pl.ds(base + k * BLK, BLK)
        pltpu.make_async_copy(buf.at[b], out_hbm.at[c, sl], sems[b]).wait()


DEPT = N_EDGES // NW
DSB = 80
DNB = DEPT // DSB


@functools.partial(
    pl.kernel,
    mesh=_mesh,
    out_type=jax.ShapeDtypeStruct((NC, NPAD, D), jnp.float32),
    scratch_types=[
        pltpu.VMEM_SHARED((NPAD, D), jnp.float32),
        pltpu.VMEM((DBLK, D), jnp.float32),
        pltpu.VMEM((DBLK, D), jnp.float32),
        pltpu.VMEM((DNB, DSB), jnp.int32),
        [pltpu.SemaphoreType.DMA] * 2,
    ],
)
def _deg_kernel(ones_hbm, dst_hbm, out_hbm, acc, buf, buf2, dst_v, dsems):
    c = lax.axis_index("c")
    s = lax.axis_index("s")
    base = s * RPT
    pltpu.sync_copy(ones_hbm, buf)
    for k in range(RPT // DBLK):
        pltpu.sync_copy(buf, acc.at[pl.ds(base + k * DBLK, DBLK)])
    pltpu.sync_copy(dst_hbm.at[c * NS + s], dst_v)
    plsc.subcore_barrier()

    @pl.loop(0, DNB)
    def _edges(j):
        pltpu.sync_copy(buf.at[pl.ds(0, DSB)], acc.at[dst_v.at[j]], add=True)

    plsc.subcore_barrier()
    bufs = (buf, buf2)
    nk = RPT // DBLK
    for k in range(nk):
        b = k % 2
        sl = pl.ds(base + k * DBLK, DBLK)
        if k >= 2:
            psl = pl.ds(base + (k - 2) * DBLK, DBLK)
            pltpu.make_async_copy(bufs[b], out_hbm.at[c, psl], dsems[b]).wait()
        pltpu.sync_copy(acc.at[sl], bufs[b])
        pltpu.async_copy(bufs[b], out_hbm.at[c, sl], dsems[b])
    for k in range(nk - 2, nk):
        b = k % 2
        sl = pl.ds(base + k * DBLK, DBLK)
        pltpu.make_async_copy(bufs[b], out_hbm.at[c, sl], dsems[b]).wait()


_ROWS = 1024
_GRID = NPAD // _ROWS


def _prep0_body(x_ref, w_ref, xw_ref):
    xw_ref[...] = jnp.dot(x_ref[...], w_ref[...], preferred_element_type=jnp.float32)


def _prep1_body(degp_ref, xw_ref, xs_ref, dinv_ref):
    deg = degp_ref[0, :, 0] + degp_ref[1, :, 0] - 1.0
    dinv = lax.rsqrt(deg)
    xs_ref[...] = xw_ref[...] * dinv[:, None]
    dinv_ref[...] = jnp.broadcast_to(dinv[:, None], dinv_ref.shape)


def _mid_body(dinv_ref, p_ref, xs_ref, b_ref, w_ref, out_ref):
    dinv = dinv_ref[:, 0]
    tot = p_ref[0] + p_ref[1] - xs_ref[...]
    h = jnp.maximum(tot * dinv[:, None] + b_ref[...], 0.0)
    out_ref[...] = jnp.dot(h, w_ref[...], preferred_element_type=jnp.float32) * dinv[:, None]


def _final_body(dinv_ref, p_ref, xs_ref, b_ref, out_ref):
    dinv = dinv_ref[:, 0]
    tot = p_ref[0] + p_ref[1] - xs_ref[...]
    out_ref[...] = tot * dinv[:, None] + b_ref[...]


_rows_spec = pl.BlockSpec((_ROWS, D), lambda i: (i, 0))
_dinv_spec = pl.BlockSpec((_ROWS, 16), lambda i: (i, 0))
_part_spec = pl.BlockSpec((NC, _ROWS, D), lambda i: (0, i, 0))
_degp_spec = pl.BlockSpec((NC, _ROWS, D), lambda i: (0, i, 0))
_w_spec = pl.BlockSpec((D, D), lambda i: (0, 0))
_b_spec = pl.BlockSpec((1, D), lambda i: (0, 0))
_f32 = lambda shape: jax.ShapeDtypeStruct(shape, jnp.float32)

_prep0 = pl.pallas_call(
    _prep0_body,
    grid=(_GRID,),
    in_specs=[_rows_spec, _w_spec],
    out_specs=_rows_spec,
    out_shape=_f32((NPAD, D)),
)

_prep1 = pl.pallas_call(
    _prep1_body,
    grid=(_GRID,),
    in_specs=[_degp_spec, _rows_spec],
    out_specs=[_rows_spec, _dinv_spec],
    out_shape=[_f32((NPAD, D)), _f32((NPAD, 16))],
)

_mid = pl.pallas_call(
    _mid_body,
    grid=(_GRID,),
    in_specs=[_dinv_spec, _part_spec, _rows_spec, _b_spec, _w_spec],
    out_specs=_rows_spec,
    out_shape=_f32((NPAD, D)),
)

_final = pl.pallas_call(
    _final_body,
    grid=(_GRID,),
    in_specs=[_dinv_spec, _part_spec, _rows_spec, _b_spec],
    out_specs=_rows_spec,
    out_shape=_f32((NPAD, D)),
)


@jax.jit
def kernel(x, edge_index, W1, b1, W2, b2):
    ei = edge_index.astype(jnp.int32)
    npadrows = NPAD - N_NODES
    pad_idx = N_NODES + (jnp.arange(EPAD - N_EDGES, dtype=jnp.int32) % npadrows)
    src = jnp.concatenate([ei[0], pad_idx]).reshape(NC, NS, NB, BLK)
    dst = jnp.concatenate([ei[1], pad_idx]).reshape(NC, NS, NB, BLK)
    dst_d = ei[1].reshape(NW, DNB, DSB)
    x_pad = jnp.pad(x, ((0, npadrows), (0, 0)))
    ones_tab = jnp.ones((DBLK, D), jnp.float32)

    degp = _deg_kernel(ones_tab, dst_d)
    xw1 = _prep0(x_pad, W1)
    xs1, dinv = _prep1(degp, xw1)
    p1 = _scatter_kernel(xs1, src, dst)
    xs2 = _mid(dinv, p1, xs1, b1.reshape(1, D), W2)
    p2 = _scatter_kernel(xs2, src, dst)
    out = _final(dinv, p2, xs2, b2.reshape(1, D))
    return out[:N_NODES]


# speedup vs baseline: 1.1580x; 1.0820x over previous
import functools

import jax
import jax.numpy as jnp
from jax import lax
from jax.experimental import pallas as pl
from jax.experimental.pallas import tpu as pltpu
from jax.experimental.pallas import tpu_sc as plsc

N_NODES = 10000
D = 128
N_EDGES = 320000

NC = 2
NS = 16
NW = NC * NS

RPT = 640
NPAD = NS * RPT
BLK = 64
NB = 160
NBC = 16
NBUF = 4
DBLK = 128
EPT = NB * BLK
EPAD = NW * EPT

_mesh = plsc.VectorSubcoreMesh(core_axis_name="c", subcore_axis_name="s")


@functools.partial(
    pl.kernel,
    mesh=_mesh,
    out_type=jax.ShapeDtypeStruct((NC, NPAD, D), jnp.float32),
    scratch_types=[
        pltpu.VMEM_SHARED((NPAD, D), jnp.float32),
        pltpu.VMEM((NBUF, BLK, D), jnp.float32),
        pltpu.VMEM((2 * NBC, BLK), jnp.int32),
        pltpu.VMEM((2 * NBC, BLK), jnp.int32),
        [pltpu.SemaphoreType.DMA] * NBUF,
    ],
)
def _scatter_kernel(xs_hbm, src_hbm, dst_hbm, out_hbm, acc, buf, src_v, dst_v, sems):
    c = lax.axis_index("c")
    s = lax.axis_index("s")
    base = s * RPT
    NIK = RPT // BLK
    for b in range(min(NBUF, NIK)):
        pltpu.async_copy(xs_hbm.at[pl.ds(base + b * BLK, BLK)], buf.at[b], sems[b])
    for k in range(NIK):
        b = k % NBUF
        sl = pl.ds(base + k * BLK, BLK)
        pltpu.make_async_copy(xs_hbm.at[sl], buf.at[b], sems[b]).wait()
        pltpu.sync_copy(buf.at[b], acc.at[sl])
        if k + NBUF < NIK:
            nsl = pl.ds(base + (k + NBUF) * BLK, BLK)
            pltpu.async_copy(xs_hbm.at[nsl], buf.at[b], sems[b])
    plsc.subcore_barrier()

    pltpu.sync_copy(src_hbm.at[c, s, pl.ds(0, NBC)], src_v.at[pl.ds(0, NBC)])
    pltpu.sync_copy(dst_hbm.at[c, s, pl.ds(0, NBC)], dst_v.at[pl.ds(0, NBC)])
    for b in range(NBUF - 1):
        pltpu.async_copy(xs_hbm.at[src_v.at[b]], buf.at[b], sems[b])

    @pl.loop(0, NB, step=NBC)
    def _chunk(j0):
        q = j0 // NBC
        p = lax.rem(q, 2) * NBC
        pn = lax.rem(q + 1, 2) * NBC

        @pl.when(j0 + NBC < NB)
        def _():
            pltpu.sync_copy(src_hbm.at[c, s, pl.ds(j0 + NBC, NBC)], src_v.at[pl.ds(pn, NBC)])
            pltpu.sync_copy(dst_hbm.at[c, s, pl.ds(j0 + NBC, NBC)], dst_v.at[pl.ds(pn, NBC)])

        for i in range(NBC):
            bi = i % NBUF
            pltpu.make_async_copy(xs_hbm.at[src_v.at[p + i]], buf.at[bi], sems[bi]).wait()
            nxt = i + NBUF - 1
            bn = nxt % NBUF
            if nxt < NBC:
                pltpu.async_copy(xs_hbm.at[src_v.at[p + nxt]], buf.at[bn], sems[bn])
            else:

                @pl.when(j0 + NBC < NB)
                def _():
                    pltpu.async_copy(xs_hbm.at[src_v.at[pn + nxt - NBC]], buf.at[bn], sems[bn])

            pltpu.sync_copy(buf.at[bi], acc.at[dst_v.at[p + i]], add=True)

    plsc.subcore_barrier()
    for k in range(NIK):
        b = k % NBUF
        sl = pl.ds(base + k * BLK, BLK)
        if k >= NBUF:
            psl = pl.ds(base + (k - NBUF) * BLK, BLK)
            pltpu.make_async_copy(buf.at[b], out_hbm.at[c, psl], sems[b]).wait()
        pltpu.sync_copy(acc.at[sl], buf.at[b])
        pltpu.async_copy(buf.at[b], out_hbm.at[c, sl], sems[b])
    for k in range(NIK - NBUF, NIK):
        b = k % NBUF
        sl = pl.ds(base + k * BLK, BLK)
        pltpu.make_async_copy(buf.at[b], out_hbm.at[c, sl], sems[b]).wait()


DEPT = N_EDGES // NW
DSB = 80
DNB = DEPT // DSB


@functools.partial(
    pl.kernel,
    mesh=_mesh,
    out_type=jax.ShapeDtypeStruct((NC, NPAD, D), jnp.float32),
    scratch_types=[
        pltpu.VMEM_SHARED((NPAD, D), jnp.float32),
        pltpu.VMEM((DBLK, D), jnp.float32),
        pltpu.VMEM((DBLK, D), jnp.float32),
        pltpu.VMEM((DNB, DSB), jnp.int32),
        [pltpu.SemaphoreType.DMA] * 2,
    ],
)
def _deg_kernel(ones_hbm, dst_hbm, out_hbm, acc, buf, buf2, dst_v, dsems):
    c = lax.axis_index("c")
    s = lax.axis_index("s")
    base = s * RPT
    pltpu.sync_copy(ones_hbm, buf)
    for k in range(RPT // DBLK):
        pltpu.sync_copy(buf, acc.at[pl.ds(base + k * DBLK, DBLK)])
    pltpu.sync_copy(dst_hbm.at[c * NS + s], dst_v)
    plsc.subcore_barrier()

    @pl.loop(0, DNB)
    def _edges(j):
        pltpu.sync_copy(buf.at[pl.ds(0, DSB)], acc.at[dst_v.at[j]], add=True)

    plsc.subcore_barrier()
    bufs = (buf, buf2)
    nk = RPT // DBLK
    for k in range(nk):
        b = k % 2
        sl = pl.ds(base + k * DBLK, DBLK)
        if k >= 2:
            psl = pl.ds(base + (k - 2) * DBLK, DBLK)
            pltpu.make_async_copy(bufs[b], out_hbm.at[c, psl], dsems[b]).wait()
        pltpu.sync_copy(acc.at[sl], bufs[b])
        pltpu.async_copy(bufs[b], out_hbm.at[c, sl], dsems[b])
    for k in range(nk - 2, nk):
        b = k % 2
        sl = pl.ds(base + k * DBLK, DBLK)
        pltpu.make_async_copy(bufs[b], out_hbm.at[c, sl], dsems[b]).wait()


_ROWS = 1024
_GRID = NPAD // _ROWS


def _prep0_body(x_ref, w_ref, xw_ref):
    xw_ref[...] = jnp.dot(x_ref[...], w_ref[...], preferred_element_type=jnp.float32)


def _prep1_body(degp_ref, xw_ref, xs_ref, dinv_ref):
    deg = degp_ref[0, :, 0] + degp_ref[1, :, 0] - 1.0
    dinv = lax.rsqrt(deg)
    xs_ref[...] = xw_ref[...] * dinv[:, None]
    dinv_ref[...] = jnp.broadcast_to(dinv[:, None], dinv_ref.shape)


def _mid_body(dinv_ref, p_ref, xs_ref, b_ref, w_ref, out_ref):
    dinv = dinv_ref[:, 0]
    tot = p_ref[0] + p_ref[1] - xs_ref[...]
    h = jnp.maximum(tot * dinv[:, None] + b_ref[...], 0.0)
    out_ref[...] = jnp.dot(h, w_ref[...], preferred_element_type=jnp.float32) * dinv[:, None]


def _final_body(dinv_ref, p_ref, xs_ref, b_ref, out_ref):
    dinv = dinv_ref[:, 0]
    tot = p_ref[0] + p_ref[1] - xs_ref[...]
    out_ref[...] = tot * dinv[:, None] + b_ref[...]


_rows_spec = pl.BlockSpec((_ROWS, D), lambda i: (i, 0))
_dinv_spec = pl.BlockSpec((_ROWS, 16), lambda i: (i, 0))
_part_spec = pl.BlockSpec((NC, _ROWS, D), lambda i: (0, i, 0))
_degp_spec = pl.BlockSpec((NC, _ROWS, D), lambda i: (0, i, 0))
_w_spec = pl.BlockSpec((D, D), lambda i: (0, 0))
_b_spec = pl.BlockSpec((1, D), lambda i: (0, 0))
_f32 = lambda shape: jax.ShapeDtypeStruct(shape, jnp.float32)

_prep0 = pl.pallas_call(
    _prep0_body,
    grid=(_GRID,),
    in_specs=[_rows_spec, _w_spec],
    out_specs=_rows_spec,
    out_shape=_f32((NPAD, D)),
)

_prep1 = pl.pallas_call(
    _prep1_body,
    grid=(_GRID,),
    in_specs=[_degp_spec, _rows_spec],
    out_specs=[_rows_spec, _dinv_spec],
    out_shape=[_f32((NPAD, D)), _f32((NPAD, 16))],
)

_mid = pl.pallas_call(
    _mid_body,
    grid=(_GRID,),
    in_specs=[_dinv_spec, _part_spec, _rows_spec, _b_spec, _w_spec],
    out_specs=_rows_spec,
    out_shape=_f32((NPAD, D)),
)

_final = pl.pallas_call(
    _final_body,
    grid=(_GRID,),
    in_specs=[_dinv_spec, _part_spec, _rows_spec, _b_spec],
    out_specs=_rows_spec,
    out_shape=_f32((NPAD, D)),
)


@jax.jit
def kernel(x, edge_index, W1, b1, W2, b2):
    ei = edge_index.astype(jnp.int32)
    npadrows = NPAD - N_NODES
    pad_idx = N_NODES + (jnp.arange(EPAD - N_EDGES, dtype=jnp.int32) % npadrows)
    src = jnp.concatenate([ei[0], pad_idx]).reshape(NC, NS, NB, BLK)
    dst = jnp.concatenate([ei[1], pad_idx]).reshape(NC, NS, NB, BLK)
    dst_d = ei[1].reshape(NW, DNB, DSB)
    x_pad = jnp.pad(x, ((0, npadrows), (0, 0)))
    ones_tab = jnp.ones((DBLK, D), jnp.float32)

    degp = _deg_kernel(ones_tab, dst_d)
    xw1 = _prep0(x_pad, W1)
    xs1, dinv = _prep1(degp, xw1)
    p1 = _scatter_kernel(xs1, src, dst)
    xs2 = _mid(dinv, p1, xs1, b1.reshape(1, D), W2)
    p2 = _scatter_kernel(xs2, src, dst)
    out = _final(dinv, p2, xs2, b2.reshape(1, D))
    return out[:N_NODES]
